# baseline (speedup 1.0000x reference)
import functools

import jax
import jax.numpy as jnp
from jax import lax
from jax.experimental import pallas as pl
from jax.experimental.pallas import tpu as pltpu
from jax.experimental.pallas import tpu_sc as plsc

N_COLS = 22
VOCAB = 100000
EMB_DIM = 32
BATCH = 16384

NC = 2
NS = 16
L = 16
NW = NC * NS

TOTAL = BATCH * N_COLS
PER_TILE = TOTAL // NW
CHUNK = 512
N_CHUNKS = PER_TILE // CHUNK
IDX_ROWS = CHUNK // 128


def _sc_body(feat_hbm, mean_hbm, std_hbm, out_hbm, idx_v, mbuf, sbuf, sem):
    wid = lax.axis_index("s") * NC + lax.axis_index("c")
    base = wid * PER_TILE
    rowbase = wid * (PER_TILE // 128)
    iota = lax.broadcasted_iota(jnp.int32, (L,), 0)

    @pl.loop(0, N_CHUNKS)
    def chunk_loop(g):
        cbase = base + g * CHUNK
        pltpu.sync_copy(feat_hbm.at[pl.ds(rowbase + g * IDX_ROWS, IDX_ROWS)],
                        idx_v)
        for r in range(IDX_ROWS):
            for j in range(128 // L):
                sl = pl.ds(j * L, L)
                pos = (cbase + r * 128 + j * L) + iota
                off = lax.rem(pos, N_COLS) * VOCAB
                idx_v[r, sl] = idx_v[r, sl] + off
        cps = []
        for r in range(IDX_ROWS):
            dst = pl.ds(r * 128, 128)
            cps.append(pltpu.async_copy(mean_hbm.at[idx_v.at[r]],
                                        mbuf.at[dst], sem))
            cps.append(pltpu.async_copy(std_hbm.at[idx_v.at[r]],
                                        sbuf.at[dst], sem))
        for cp in cps:
            cp.wait()
        pltpu.sync_copy(mbuf, out_hbm.at[pl.ds(cbase, CHUNK), 0])
        pltpu.sync_copy(sbuf, out_hbm.at[pl.ds(cbase, CHUNK), 1])


@jax.jit
def kernel(features, emb_mean, emb_std):
    feat = features.astype(jnp.int32).reshape(TOTAL // 128, 128)
    mean_flat = emb_mean.reshape(N_COLS * VOCAB, EMB_DIM)
    std_flat = emb_std.reshape(N_COLS * VOCAB, EMB_DIM)
    run = pl.kernel(
        _sc_body,
        out_type=jax.ShapeDtypeStruct((TOTAL, 2, EMB_DIM), jnp.float32),
        mesh=plsc.VectorSubcoreMesh(core_axis_name="c", subcore_axis_name="s"),
        scratch_types=[
            pltpu.VMEM((IDX_ROWS, 128), jnp.int32),
            pltpu.VMEM((CHUNK, EMB_DIM), jnp.float32),
            pltpu.VMEM((CHUNK, EMB_DIM), jnp.float32),
            pltpu.SemaphoreType.DMA,
        ],
        compiler_params=pltpu.CompilerParams(use_tc_tiling_on_sc=False),
    )
    out = run(feat, mean_flat, std_flat)
    return out.reshape(BATCH, N_COLS, 2 * EMB_DIM)


# speedup vs baseline: 3.0910x; 3.0910x over previous
import jax
import jax.numpy as jnp
from jax import lax
from jax.experimental import pallas as pl
from jax.experimental.pallas import tpu as pltpu
from jax.experimental.pallas import tpu_sc as plsc

N_COLS = 22
VOCAB = 100000
EMB_DIM = 32
BATCH = 16384

NC = 2
NS = 16
L = 16

OUT_CHUNK = 8192


def _sc_body(feat_hbm, mean_hbm, std_hbm, out_hbm, featv, tabv, outv, sem):
    wid = lax.axis_index("s") * NC + lax.axis_index("c")
    half = wid // 16
    lane = wid % 16

    def do_pair(tab_hbm, t, j):
        ee = 2 * lane + j
        eo = half * EMB_DIM + ee
        pltpu.sync_copy(
            tab_hbm.at[t * 4 + lax.shift_right_logical(ee, 3),
                       lax.bitwise_and(ee, 7)],
            tabv)
        orow = t * 8 + lax.shift_right_logical(eo, 3)
        osub = lax.bitwise_and(eo, 7)
        for c in range(BATCH // OUT_CHUNK):
            @pl.loop(0, OUT_CHUNK, step=L, unroll=8)
            def g_loop(g):
                idx = featv[pl.ds(c * OUT_CHUNK + g, L)]
                outv[pl.ds(g, L)] = plsc.load_gather(tabv, [idx])
            pltpu.sync_copy(outv,
                            out_hbm.at[orow, osub, pl.ds(c * OUT_CHUNK,
                                                         OUT_CHUNK)])

    for t in range(N_COLS):
        pltpu.sync_copy(feat_hbm.at[t], featv)
        for j in range(2):
            @pl.when(half == 0)
            def _():
                do_pair(mean_hbm, t, j)

            @pl.when(half == 1)
            def _():
                do_pair(std_hbm, t, j)


@jax.jit
def kernel(features, emb_mean, emb_std):
    feat = features.astype(jnp.int32).T
    mean_t = emb_mean.transpose(0, 2, 1).reshape(N_COLS * 4, 8, VOCAB)
    std_t = emb_std.transpose(0, 2, 1).reshape(N_COLS * 4, 8, VOCAB)
    run = pl.kernel(
        _sc_body,
        out_type=jax.ShapeDtypeStruct((N_COLS * 8, 8, BATCH), jnp.float32),
        mesh=plsc.VectorSubcoreMesh(core_axis_name="c", subcore_axis_name="s"),
        scratch_types=[
            pltpu.VMEM((BATCH,), jnp.int32),
            pltpu.VMEM((VOCAB,), jnp.float32),
            pltpu.VMEM((OUT_CHUNK,), jnp.float32),
            pltpu.SemaphoreType.DMA,
        ],
        compiler_params=pltpu.CompilerParams(use_tc_tiling_on_sc=True,
                                             needs_layout_passes=False),
    )
    out = run(feat, mean_t, std_t)
    return out.reshape(N_COLS, 2 * EMB_DIM, BATCH).transpose(2, 0, 1)


# speedup vs baseline: 3.1165x; 1.0082x over previous
import jax
import jax.numpy as jnp
from jax import lax
from jax.experimental import pallas as pl
from jax.experimental.pallas import tpu as pltpu
from jax.experimental.pallas import tpu_sc as plsc

N_COLS = 22
VOCAB = 100000
EMB_DIM = 32
BATCH = 16384

NC = 2
NS = 16
L = 16

CHUNKS = ((0, 7168), (7168, 7168), (14336, 2048))


def _sc_body(feat_hbm, mean_hbm, std_hbm, out_hbm, featv, tabv, outv,
             rowsem, outsem):
    wid = lax.axis_index("s") * NC + lax.axis_index("c")
    d0sub = lax.shift_right_logical(wid, 3)
    d1 = lax.bitwise_and(wid, 7)

    pairs = [(t, which) for t in range(N_COLS) for which in (0, 1)]

    def fire_row(t, which):
        src = mean_hbm if which == 0 else std_hbm
        return pltpu.async_copy(src.at[t * 4 + d0sub, d1], tabv, rowsem)

    pltpu.sync_copy(feat_hbm.at[0], featv)
    row_cp = fire_row(*pairs[0])
    row_cp.wait()

    pending = []
    slot = 0
    for p, (t, which) in enumerate(pairs):
        eo = wid + which * EMB_DIM
        orow = t * 8 + lax.shift_right_logical(eo, 3)
        osub = lax.bitwise_and(eo, 7)
        for k, (off, size) in enumerate(CHUNKS):
            if len(pending) >= 2:
                pending.pop(0).wait()

            @pl.loop(0, size, step=L, unroll=8)
            def g_loop(g, off=off, slot=slot):
                idx = featv[pl.ds(off + g, L)]
                outv[slot, pl.ds(g, L)] = plsc.load_gather(tabv, [idx])

            if k == len(CHUNKS) - 1 and p + 1 < len(pairs):
                tn, wn = pairs[p + 1]
                if tn != t:
                    pltpu.sync_copy(feat_hbm.at[tn], featv)
                row_cp = fire_row(tn, wn)
            pending.append(
                pltpu.async_copy(outv.at[slot, pl.ds(0, size)],
                                 out_hbm.at[orow, osub, pl.ds(off, size)],
                                 outsem))
            slot = 1 - slot
        if p + 1 < len(pairs):
            row_cp.wait()
    for cp in pending:
        cp.wait()


@jax.jit
def kernel(features, emb_mean, emb_std):
    feat = features.astype(jnp.int32).T
    mean_t = emb_mean.transpose(0, 2, 1).reshape(N_COLS * 4, 8, VOCAB)
    std_t = emb_std.transpose(0, 2, 1).reshape(N_COLS * 4, 8, VOCAB)
    run = pl.kernel(
        _sc_body,
        out_type=jax.ShapeDtypeStruct((N_COLS * 8, 8, BATCH), jnp.float32),
        mesh=plsc.VectorSubcoreMesh(core_axis_name="c", subcore_axis_name="s"),
        scratch_types=[
            pltpu.VMEM((BATCH,), jnp.int32),
            pltpu.VMEM((VOCAB,), jnp.float32),
            pltpu.VMEM((2, 7168), jnp.float32),
            pltpu.SemaphoreType.DMA,
            pltpu.SemaphoreType.DMA,
        ],
        compiler_params=pltpu.CompilerParams(use_tc_tiling_on_sc=True,
                                             needs_layout_passes=False),
    )
    out = run(feat, mean_t, std_t)
    return out.reshape(N_COLS, 2 * EMB_DIM, BATCH).transpose(2, 0, 1)


# speedup vs baseline: 5.8767x; 1.8857x over previous
import jax
import jax.numpy as jnp
from jax import lax
from jax.experimental import pallas as pl
from jax.experimental.pallas import tpu as pltpu
from jax.experimental.pallas import tpu_sc as plsc

N_COLS = 22
VOCAB = 100000
EMB_DIM = 32
BATCH = 16384

NC = 2
NS = 16
L = 16

CHUNKS = ((0, 7168), (7168, 7168), (14336, 2048))


def _sc_body(feat_hbm, mean_hbm, std_hbm, out_hbm, featv, tabv, outv,
             rowsem, outsem):
    wid = lax.axis_index("s") * NC + lax.axis_index("c")
    d0sub = lax.shift_right_logical(wid, 3)
    d1 = lax.bitwise_and(wid, 7)

    pairs = [(t, which) for t in range(N_COLS) for which in (0, 1)]

    def fire_row(t, which):
        src = mean_hbm if which == 0 else std_hbm
        return pltpu.async_copy(src.at[t * 4 + d0sub, d1], tabv, rowsem)

    pltpu.sync_copy(feat_hbm.at[0], featv)
    row_cp = fire_row(*pairs[0])
    row_cp.wait()

    pending = []
    slot = 0
    for p, (t, which) in enumerate(pairs):
        eo = wid + which * EMB_DIM
        orow = t * 8 + lax.shift_right_logical(eo, 3)
        osub = lax.bitwise_and(eo, 7)
        for k, (off, size) in enumerate(CHUNKS):
            if len(pending) >= 2:
                pending.pop(0).wait()

            @plsc.parallel_loop(0, size, step=L, unroll=8)
            def g_loop(g, off=off, slot=slot):
                idx = featv[pl.ds(off + g, L)]
                outv[slot, pl.ds(g, L)] = plsc.load_gather(tabv, [idx])

            if k == len(CHUNKS) - 1 and p + 1 < len(pairs):
                tn, wn = pairs[p + 1]
                if tn != t:
                    pltpu.sync_copy(feat_hbm.at[tn], featv)
                row_cp = fire_row(tn, wn)
            pending.append(
                pltpu.async_copy(outv.at[slot, pl.ds(0, size)],
                                 out_hbm.at[orow, osub, pl.ds(off, size)],
                                 outsem))
            slot = 1 - slot
        if p + 1 < len(pairs):
            row_cp.wait()
    for cp in pending:
        cp.wait()


@jax.jit
def kernel(features, emb_mean, emb_std):
    feat = features.astype(jnp.int32).T
    mean_t = emb_mean.transpose(0, 2, 1).reshape(N_COLS * 4, 8, VOCAB)
    std_t = emb_std.transpose(0, 2, 1).reshape(N_COLS * 4, 8, VOCAB)
    run = pl.kernel(
        _sc_body,
        out_type=jax.ShapeDtypeStruct((N_COLS * 8, 8, BATCH), jnp.float32),
        mesh=plsc.VectorSubcoreMesh(core_axis_name="c", subcore_axis_name="s"),
        scratch_types=[
            pltpu.VMEM((BATCH,), jnp.int32),
            pltpu.VMEM((VOCAB,), jnp.float32),
            pltpu.VMEM((2, 7168), jnp.float32),
            pltpu.SemaphoreType.DMA,
            pltpu.SemaphoreType.DMA,
        ],
        compiler_params=pltpu.CompilerParams(use_tc_tiling_on_sc=True,
                                             needs_layout_passes=False),
    )
    out = run(feat, mean_t, std_t)
    return out.reshape(N_COLS, 2 * EMB_DIM, BATCH).transpose(2, 0, 1)
